# baseline (speedup 1.0000x reference)
import jax
import jax.numpy as jnp
from jax.experimental import pallas as pl
from jax.experimental.pallas import tpu as pltpu


def _body(ids_ref, x_ref, out_ref):
    w = (ids_ref[0] != 0).astype(jnp.float32)
    s = jax.lax.dot_general(
        w, x_ref[0], (((1,), (0,)), ((), ())),
        preferred_element_type=jnp.float32)
    cnt = jnp.sum(w)
    out_ref[...] = (s / cnt).reshape(1, 8, 128)


def kernel(x, input):
    B, S, D = x.shape
    ids = input.reshape(B, 1, S).astype(jnp.int32)
    out = pl.pallas_call(
        _body,
        grid=(B,),
        in_specs=[
            pl.BlockSpec((1, 1, S), lambda b: (b, 0, 0)),
            pl.BlockSpec((1, S, D), lambda b: (b, 0, 0)),
        ],
        out_specs=pl.BlockSpec((1, 8, D // 8), lambda b: (b, 0, 0)),
        out_shape=jax.ShapeDtypeStruct((B, 8, D // 8), jnp.float32),
    )(ids, x)
    return out.reshape(B, D)
